# baseline (speedup 1.0000x reference)
import jax
import jax.numpy as jnp
from jax import lax
from jax.experimental import pallas as pl
from jax.experimental.pallas import tpu as pltpu
from jax.experimental.pallas import tpu_sc as plsc

BATCH = 4096
HIST = 200
WORD_DIM = 32
B = BATCH * HIST
NW = 32
B_CHUNK = 1024
UNITS = (BATCH // B_CHUNK) * HIST
UNITS_PER_W = UNITS // NW
QPH = BATCH // B_CHUNK
TR_STRIDE = B_CHUNK + 1


def _emb_body(idx_hbm, table_hbm, out_hbm, idx_v, rows_v, tr_v, sem):
    wid = lax.axis_index("s") * 2 + lax.axis_index("c")
    lanes = lax.iota(jnp.int32, 16)
    c_lo = lanes
    c_hi = lanes + 16

    def step(u, _):
        unit = wid * UNITS_PER_W + u
        h = unit // QPH
        b0 = (unit % QPH) * B_CHUNK
        off = pl.multiple_of(h * BATCH + b0, B_CHUNK)
        pltpu.sync_copy(idx_hbm.at[pl.ds(off, B_CHUNK)], idx_v)
        pltpu.async_copy(table_hbm.at[idx_v], rows_v, sem).wait()

        @plsc.parallel_loop(0, B_CHUNK, unroll=8)
        def _(i):
            b_idx = jnp.full((16,), 0, jnp.int32) + i
            v0 = rows_v[i, pl.ds(0, 16)]
            v1 = rows_v[i, pl.ds(16, 16)]
            plsc.store_scatter(tr_v, [c_lo, b_idx], v0)
            plsc.store_scatter(tr_v, [c_hi, b_idx], v1)

        for c in range(WORD_DIM):
            pltpu.sync_copy(
                tr_v.at[c, pl.ds(0, B_CHUNK)],
                out_hbm.at[h, c, pl.ds(b0, B_CHUNK)],
            )
        return 0

    lax.fori_loop(0, UNITS_PER_W, step, 0)


V = 1000000
R_CH = 1024
N_FULL = V // R_CH
N_CH = N_FULL + 1
TAIL_R0 = V - R_CH
CH_PER_W = -(-N_CH // NW)


def _tr_body(wem_t_hbm, out_hbm, buf, outbuf, sem):
    wid = lax.axis_index("s") * 2 + lax.axis_index("c")
    lanes = lax.iota(jnp.int32, 16)

    def chunk(k, _):
        cid = wid + k * NW

        @pl.when(cid < N_CH)
        def _():
            r0 = jnp.where(cid == N_FULL, TAIL_R0, cid * R_CH)
            r0 = pl.multiple_of(r0, 8)
            for c in range(WORD_DIM):
                pltpu.async_copy(
                    wem_t_hbm.at[c, pl.ds(r0, R_CH)], buf.at[c], sem
                )
            for c in range(WORD_DIM):
                pltpu.make_async_copy(
                    wem_t_hbm.at[c, pl.ds(r0, R_CH)], buf.at[c], sem
                ).wait()

            @plsc.parallel_loop(0, R_CH // 16, unroll=4)
            def _(r16):
                rvec = r16 * 16 + lanes
                for d in range(16):
                    cvec = (d + lanes) & 15
                    v = plsc.load_gather(buf, [cvec, rvec])
                    plsc.store_scatter(outbuf, [rvec, cvec], v)
                    v = plsc.load_gather(buf, [cvec + 16, rvec])
                    plsc.store_scatter(outbuf, [rvec, cvec + 16], v)

            pltpu.sync_copy(outbuf, out_hbm.at[pl.ds(r0, R_CH), :])

        return 0

    lax.fori_loop(0, CH_PER_W, chunk, 0)


@jax.jit
def _table_rm(wem_t):
    return pl.kernel(
        _tr_body,
        out_type=jax.ShapeDtypeStruct((V, WORD_DIM), jnp.float32),
        mesh=plsc.VectorSubcoreMesh(core_axis_name="c", subcore_axis_name="s"),
        scratch_types=[
            pltpu.VMEM((WORD_DIM, R_CH), jnp.float32),
            pltpu.VMEM((R_CH, WORD_DIM), jnp.float32),
            pltpu.SemaphoreType.DMA,
        ],
        compiler_params=pltpu.CompilerParams(
            use_tc_tiling_on_sc=False, needs_layout_passes=False
        ),
    )(wem_t)


@jax.jit
def _emb(idx, word_em):
    return pl.kernel(
        _emb_body,
        out_type=jax.ShapeDtypeStruct((HIST, WORD_DIM, BATCH), jnp.float32),
        mesh=plsc.VectorSubcoreMesh(core_axis_name="c", subcore_axis_name="s"),
        scratch_types=[
            pltpu.VMEM((B_CHUNK,), jnp.int32),
            pltpu.VMEM((B_CHUNK, WORD_DIM), jnp.float32),
            pltpu.VMEM((WORD_DIM, TR_STRIDE), jnp.float32),
            pltpu.SemaphoreType.DMA,
        ],
        compiler_params=pltpu.CompilerParams(
            use_tc_tiling_on_sc=False, needs_layout_passes=False
        ),
    )(idx, word_em)


def kernel(review, word_em):
    idx = review.T.reshape(B).astype(jnp.int32)
    table_rm = _table_rm(word_em.T)
    out = _emb(idx, table_rm)
    return jnp.transpose(out, (2, 0, 1))


# speedup vs baseline: 6.1950x; 6.1950x over previous
import jax
import jax.numpy as jnp
from jax import lax
from jax.experimental import pallas as pl
from jax.experimental.pallas import tpu as pltpu
from jax.experimental.pallas import tpu_sc as plsc

BATCH = 4096
HIST = 200
WORD_DIM = 32
B = BATCH * HIST
NW = 32
B_CHUNK = 1024
UNITS = (BATCH // B_CHUNK) * HIST
UNITS_PER_W = UNITS // NW
QPH = BATCH // B_CHUNK
TR_STRIDE = B_CHUNK + 1


def _emb_body(idx_hbm, table_hbm, out_hbm, idx_v, rows_v, tr_v, sem):
    wid = lax.axis_index("s") * 2 + lax.axis_index("c")
    lanes = lax.iota(jnp.int32, 16)
    c_lo = lanes
    c_hi = lanes + 16

    def step(u, _):
        unit = wid * UNITS_PER_W + u
        h = unit // QPH
        b0 = (unit % QPH) * B_CHUNK
        off = pl.multiple_of(h * BATCH + b0, B_CHUNK)
        pltpu.sync_copy(idx_hbm.at[pl.ds(off, B_CHUNK)], idx_v)
        pltpu.async_copy(table_hbm.at[idx_v], rows_v, sem).wait()

        @plsc.parallel_loop(0, B_CHUNK, unroll=8)
        def _(i):
            b_idx = jnp.full((16,), 0, jnp.int32) + i
            v0 = rows_v[i, pl.ds(0, 16)]
            v1 = rows_v[i, pl.ds(16, 16)]
            plsc.store_scatter(tr_v, [c_lo, b_idx], v0)
            plsc.store_scatter(tr_v, [c_hi, b_idx], v1)

        for c in range(WORD_DIM):
            pltpu.sync_copy(
                tr_v.at[c, pl.ds(0, B_CHUNK)],
                out_hbm.at[h, c, pl.ds(b0, B_CHUNK)],
            )
        return 0

    lax.fori_loop(0, UNITS_PER_W, step, 0)


V = 1000000
R_CH = 1024
N_FULL = V // R_CH
N_CH = N_FULL + 1
V_PAD = 1000064
TAIL_R0 = 999040
CH_PER_W = -(-N_CH // NW)
RB_STRIDE = 33


def _tr_body(wem_t_hbm, out_hbm, buf, rowbuf, outbuf, sem):
    wid = lax.axis_index("s") * 2 + lax.axis_index("c")
    lanes = lax.iota(jnp.int32, 16)

    def chunk(k, _):
        cid = wid + k * NW

        @pl.when(cid < N_CH)
        def _():
            r0 = jnp.where(cid == N_FULL, TAIL_R0, cid * R_CH)
            r0 = pl.multiple_of(r0, 128)
            for t in range(32):
                c8, rt = t // 8, t % 8
                pltpu.async_copy(
                    wem_t_hbm.at[
                        pl.ds(c8 * 8, 8), pl.ds(r0 + rt * 128, 128)
                    ],
                    buf.at[t],
                    sem,
                )
            for t in range(32):
                c8, rt = t // 8, t % 8
                pltpu.make_async_copy(
                    wem_t_hbm.at[
                        pl.ds(c8 * 8, 8), pl.ds(r0 + rt * 128, 128)
                    ],
                    buf.at[t],
                    sem,
                ).wait()

            @plsc.parallel_loop(0, 256, unroll=2)
            def _(tc):
                t = tc // 8
                cc = tc % 8
                cvec = (t // 8) * 8 + cc + jnp.zeros((16,), jnp.int32)
                rbase = (t % 8) * 128
                for bb0 in range(8):
                    rvec = rbase + bb0 * 16 + lanes
                    v = buf[t, cc, pl.ds(bb0 * 16, 16)]
                    plsc.store_scatter(rowbuf, [rvec * RB_STRIDE + cvec], v)

            @plsc.parallel_loop(0, R_CH // 16, unroll=2)
            def _(r16):
                for j in range(16):
                    line = r16 * 4 + j // 4
                    cb = (j % 4) * 32
                    rb = (r16 * 16 + j) * RB_STRIDE
                    outbuf[line, pl.ds(cb, 16)] = rowbuf[pl.ds(rb, 16)]
                    outbuf[line, pl.ds(cb + 16, 16)] = rowbuf[pl.ds(rb + 16, 16)]

            pltpu.sync_copy(
                outbuf, out_hbm.at[pl.ds(pl.multiple_of(r0 // 4, 8), R_CH // 4), :]
            )

        return 0

    lax.fori_loop(0, CH_PER_W, chunk, 0)


@jax.jit
def _table_rm(wem_t):
    return pl.kernel(
        _tr_body,
        out_type=jax.ShapeDtypeStruct((V_PAD * WORD_DIM // 128, 128), jnp.float32),
        mesh=plsc.VectorSubcoreMesh(core_axis_name="c", subcore_axis_name="s"),
        scratch_types=[
            pltpu.VMEM((32, 8, 128), jnp.float32),
            pltpu.VMEM((R_CH * RB_STRIDE,), jnp.float32),
            pltpu.VMEM((R_CH // 4, 128), jnp.float32),
            pltpu.SemaphoreType.DMA,
        ],
        compiler_params=pltpu.CompilerParams(
            use_tc_tiling_on_sc=True, needs_layout_passes=False
        ),
    )(wem_t)


@jax.jit
def _emb(idx, word_em):
    return pl.kernel(
        _emb_body,
        out_type=jax.ShapeDtypeStruct((HIST, WORD_DIM, BATCH), jnp.float32),
        mesh=plsc.VectorSubcoreMesh(core_axis_name="c", subcore_axis_name="s"),
        scratch_types=[
            pltpu.VMEM((B_CHUNK,), jnp.int32),
            pltpu.VMEM((B_CHUNK, WORD_DIM), jnp.float32),
            pltpu.VMEM((WORD_DIM, TR_STRIDE), jnp.float32),
            pltpu.SemaphoreType.DMA,
        ],
        compiler_params=pltpu.CompilerParams(
            use_tc_tiling_on_sc=False, needs_layout_passes=False
        ),
    )(idx, word_em)


def kernel(review, word_em):
    idx = review.T.reshape(B).astype(jnp.int32)
    t4 = _table_rm(word_em.T)
    table_rm = t4.reshape(V_PAD, WORD_DIM)
    out = _emb(idx, table_rm)
    return jnp.transpose(out, (2, 0, 1))


# speedup vs baseline: 8.0403x; 1.2979x over previous
import jax
import jax.numpy as jnp
from jax import lax
from jax.experimental import pallas as pl
from jax.experimental.pallas import tpu as pltpu
from jax.experimental.pallas import tpu_sc as plsc

BATCH = 4096
HIST = 200
WORD_DIM = 32
B = BATCH * HIST
NW = 32
B_CHUNK = 1024
UNITS = (BATCH // B_CHUNK) * HIST
UNITS_PER_W = UNITS // NW
QPH = BATCH // B_CHUNK
TR_STRIDE = B_CHUNK + 1


def _emb_body(idx_hbm, table_hbm, out_hbm, idx_v, rows_v, trt, sem):
    wid = lax.axis_index("s") * 2 + lax.axis_index("c")
    lanes = lax.iota(jnp.int32, 16)

    def step(u, _):
        unit = wid * UNITS_PER_W + u
        h = unit // QPH
        q = unit % QPH
        b0 = q * B_CHUNK
        off = pl.multiple_of(h * BATCH + b0, B_CHUNK)
        pltpu.sync_copy(idx_hbm.at[pl.ds(off, B_CHUNK)], idx_v)
        pltpu.async_copy(table_hbm.at[idx_v], rows_v, sem).wait()

        @plsc.parallel_loop(0, B_CHUNK // 16, unroll=4)
        def _(i16):
            ivec = i16 * 16 + lanes
            btl = i16 // 8
            bbv = (i16 * 16 - btl * 128) + lanes
            for d in range(16):
                for half in range(2):
                    cvec = half * 16 + ((d + lanes) & 15)
                    v = plsc.load_gather(rows_v, [ivec, cvec])
                    plsc.store_scatter(
                        trt,
                        [cvec >> 3, jnp.full((16,), btl, jnp.int32), cvec & 7, bbv],
                        v,
                    )

        for c8 in range(4):
            pltpu.sync_copy(
                trt.at[c8], out_hbm.at[h, c8, pl.ds(q * 8, 8)]
            )
        return 0

    lax.fori_loop(0, UNITS_PER_W, step, 0)


V = 1000000
R_CH = 1024
N_FULL = V // R_CH
N_CH = N_FULL + 1
V_PAD = 1000064
TAIL_R0 = 999040
CH_PER_W = -(-N_CH // NW)
RB_STRIDE = 33


def _tr_body(wem_t_hbm, out_hbm, buf, rowbuf, outbuf, sem):
    wid = lax.axis_index("s") * 2 + lax.axis_index("c")
    lanes = lax.iota(jnp.int32, 16)

    def chunk(k, _):
        cid = wid + k * NW

        @pl.when(cid < N_CH)
        def _():
            r0 = jnp.where(cid == N_FULL, TAIL_R0, cid * R_CH)
            r0 = pl.multiple_of(r0, 128)
            for t in range(32):
                c8, rt = t // 8, t % 8
                pltpu.async_copy(
                    wem_t_hbm.at[
                        pl.ds(c8 * 8, 8), pl.ds(r0 + rt * 128, 128)
                    ],
                    buf.at[t],
                    sem,
                )
            for t in range(32):
                c8, rt = t // 8, t % 8
                pltpu.make_async_copy(
                    wem_t_hbm.at[
                        pl.ds(c8 * 8, 8), pl.ds(r0 + rt * 128, 128)
                    ],
                    buf.at[t],
                    sem,
                ).wait()

            @plsc.parallel_loop(0, 256, unroll=2)
            def _(tc):
                t = tc // 8
                cc = tc % 8
                cvec = (t // 8) * 8 + cc + jnp.zeros((16,), jnp.int32)
                rbase = (t % 8) * 128
                for bb0 in range(8):
                    rvec = rbase + bb0 * 16 + lanes
                    v = buf[t, cc, pl.ds(bb0 * 16, 16)]
                    plsc.store_scatter(rowbuf, [rvec * RB_STRIDE + cvec], v)

            @plsc.parallel_loop(0, R_CH // 16, unroll=2)
            def _(r16):
                for j in range(16):
                    line = r16 * 4 + j // 4
                    cb = (j % 4) * 32
                    rb = (r16 * 16 + j) * RB_STRIDE
                    outbuf[line, pl.ds(cb, 16)] = rowbuf[pl.ds(rb, 16)]
                    outbuf[line, pl.ds(cb + 16, 16)] = rowbuf[pl.ds(rb + 16, 16)]

            pltpu.sync_copy(
                outbuf, out_hbm.at[pl.ds(pl.multiple_of(r0 // 4, 8), R_CH // 4), :]
            )

        return 0

    lax.fori_loop(0, CH_PER_W, chunk, 0)


@jax.jit
def _table_rm(wem_t):
    return pl.kernel(
        _tr_body,
        out_type=jax.ShapeDtypeStruct((V_PAD * WORD_DIM // 128, 128), jnp.float32),
        mesh=plsc.VectorSubcoreMesh(core_axis_name="c", subcore_axis_name="s"),
        scratch_types=[
            pltpu.VMEM((32, 8, 128), jnp.float32),
            pltpu.VMEM((R_CH * RB_STRIDE,), jnp.float32),
            pltpu.VMEM((R_CH // 4, 128), jnp.float32),
            pltpu.SemaphoreType.DMA,
        ],
        compiler_params=pltpu.CompilerParams(
            use_tc_tiling_on_sc=True, needs_layout_passes=False
        ),
    )(wem_t)


@jax.jit
def _emb(idx, word_em):
    return pl.kernel(
        _emb_body,
        out_type=jax.ShapeDtypeStruct(
            (HIST, 4, BATCH // 128, 8, 128), jnp.float32
        ),
        mesh=plsc.VectorSubcoreMesh(core_axis_name="c", subcore_axis_name="s"),
        scratch_types=[
            pltpu.VMEM((B_CHUNK,), jnp.int32),
            pltpu.VMEM((B_CHUNK, WORD_DIM), jnp.float32),
            pltpu.VMEM((4, B_CHUNK // 128, 8, 128), jnp.float32),
            pltpu.SemaphoreType.DMA,
        ],
        compiler_params=pltpu.CompilerParams(
            use_tc_tiling_on_sc=False, needs_layout_passes=False
        ),
    )(idx, word_em)


def kernel(review, word_em):
    idx = review.T.reshape(B).astype(jnp.int32)
    t4 = _table_rm(word_em.T)
    table_rm = t4.reshape(V_PAD, WORD_DIM)
    out6 = _emb(idx, table_rm)
    return out6.transpose(2, 4, 0, 1, 3).reshape(BATCH, HIST, WORD_DIM)
